# baseline (speedup 1.0000x reference)
import jax
import jax.numpy as jnp
from jax import lax
from jax.experimental import pallas as pl
from jax.experimental.pallas import tpu as pltpu
from jax.experimental.pallas import tpu_sc as plsc

F = 64
XW = 128
NBRF = 41
M = 12
N = 100000
NSEG = 2000

_BE = 2000


def _embed_body(a_ref, w_ref, b_ref, wn_ref, x_ref, y_ref):
    res = jnp.dot(a_ref[...], w_ref[...],
                  preferred_element_type=jnp.float32) + b_ref[...]
    x_ref[...] = res
    y_ref[...] = jnp.dot(res, wn_ref[...],
                         preferred_element_type=jnp.float32)


def _embed(atom_fea, w, b, wn):
    n, orig = atom_fea.shape
    return pl.pallas_call(
        _embed_body,
        grid=(n // _BE,),
        in_specs=[
            pl.BlockSpec((_BE, orig), lambda i: (i, 0)),
            pl.BlockSpec((orig, F), lambda i: (0, 0)),
            pl.BlockSpec((1, F), lambda i: (0, 0)),
            pl.BlockSpec((F, XW), lambda i: (0, 0)),
        ],
        out_specs=[pl.BlockSpec((_BE, F), lambda i: (i, 0)),
                   pl.BlockSpec((_BE, XW), lambda i: (i, 0))],
        out_shape=[jax.ShapeDtypeStruct((n, F), jnp.float32),
                   jax.ShapeDtypeStruct((n, XW), jnp.float32)],
    )(atom_fea, w, b.reshape(1, F), wn)


_NW = 32
_GCH = 160
_CH0 = 264
_CH1 = 216
_NPADG = 102400
_BPAD = _GCH * (_CH0 + _CH1) * 16
_IDXPAD = _BPAD + (_CH0 - _CH1) * _GCH


def _gather_body(table, idx_hbm, out_hbm, idx_all, rows_v,
                 g0, g1, g2, w0, w1, w2):
    cid = lax.axis_index("c")
    sid = lax.axis_index("s")
    gsem = (g0, g1, g2)
    wsem = (w0, w1, w2)

    def run(base, nch):
        pltpu.sync_copy(idx_hbm.at[pl.ds(base, _CH0 * _GCH)], idx_all)

        def gather_copy(i, b):
            return pltpu.make_async_copy(
                table.at[idx_all.at[pl.ds(i * _GCH, _GCH)]],
                rows_v.at[pl.ds(b * _GCH, _GCH)], gsem[b])

        def write_copy(i, b):
            return pltpu.make_async_copy(
                rows_v.at[pl.ds(b * _GCH, _GCH)],
                out_hbm.at[pl.ds(base + i * _GCH, _GCH)], wsem[b])

        gather_copy(0, 0).start()
        gather_copy(1, 1).start()

        def step(j, carry):
            for b in range(3):
                i = 3 * j + b
                gather_copy(i, b).wait()
                write_copy(i, b).start()
                c = (b + 2) % 3

                @pl.when(i + 2 < nch)
                def _():
                    @pl.when(i >= 1)
                    def _():
                        write_copy(i - 1, c).wait()
                    gather_copy(i + 2, c).start()
            return carry

        lax.fori_loop(0, nch // 3, step, 0)
        for t in (nch - 3, nch - 2, nch - 1):
            write_copy(t, t % 3).wait()

    @pl.when(cid == 0)
    def _():
        run(sid * (_CH0 * _GCH), _CH0)

    @pl.when(cid == 1)
    def _():
        run(16 * _CH0 * _GCH + sid * (_CH1 * _GCH), _CH1)


def _sc_gather(table, idx_pad):
    mesh = plsc.VectorSubcoreMesh(core_axis_name="c", subcore_axis_name="s")
    f = pl.kernel(
        _gather_body,
        mesh=mesh,
        out_type=jax.ShapeDtypeStruct((_BPAD, XW), jnp.float32),
        scratch_types=[
            pltpu.VMEM((_CH0 * _GCH,), jnp.int32),
            pltpu.VMEM((3 * _GCH, XW), jnp.float32),
            pltpu.SemaphoreType.DMA,
            pltpu.SemaphoreType.DMA,
            pltpu.SemaphoreType.DMA,
            pltpu.SemaphoreType.DMA,
            pltpu.SemaphoreType.DMA,
            pltpu.SemaphoreType.DMA,
        ],
    )
    return f(table, idx_pad)


_BN = 1000


def _p1_body(x_ref, g_ref, nb_ref, ws, we, bf, s_ref, q_ref):
    i = pl.program_id(0)
    s = jnp.dot(x_ref[...], ws[...],
                preferred_element_type=jnp.float32) + bf[...]
    acc_s = jnp.zeros((1, XW), jnp.float32)
    acc_q = jnp.zeros((1, XW), jnp.float32)
    for m in range(M):
        nb = nb_ref[:, pl.ds(NBRF * m, NBRF)].astype(jnp.bfloat16)
        z = s + g_ref[m] \
            + jnp.dot(nb, we[...], preferred_element_type=jnp.float32)
        acc_s += jnp.sum(z, axis=0, keepdims=True)
        acc_q += jnp.sum(z * z, axis=0, keepdims=True)

    @pl.when(i == 0)
    def _():
        s_ref[...] = jnp.zeros_like(s_ref)
        q_ref[...] = jnp.zeros_like(q_ref)

    s_ref[...] += acc_s
    q_ref[...] += acc_q


def _p2_body(x_ref, g_ref, nb_ref, ws, we, bf, k1, h1,
             ns_ref, s2_ref, q2_ref):
    i = pl.program_id(0)
    s = jnp.dot(x_ref[...], ws[...],
                preferred_element_type=jnp.float32) + bf[...]
    acc = jnp.zeros((_BN, F), jnp.float32)
    for m in range(M):
        nb = nb_ref[:, pl.ds(NBRF * m, NBRF)].astype(jnp.bfloat16)
        z = s + g_ref[m] \
            + jnp.dot(nb, we[...], preferred_element_type=jnp.float32)
        z = z * k1[...] + h1[...]
        acc += jax.nn.sigmoid(z[:, :F]) * jax.nn.softplus(z[:, F:])
    ns_ref[...] = acc

    @pl.when(i == 0)
    def _():
        s2_ref[...] = jnp.zeros_like(s2_ref)
        q2_ref[...] = jnp.zeros_like(q2_ref)

    s2_ref[...] += jnp.sum(acc, axis=0, keepdims=True)
    q2_ref[...] += jnp.sum(acc * acc, axis=0, keepdims=True)


def _conv_specs():
    xspec = pl.BlockSpec((_BN, F), lambda i: (i, 0))
    gspec = pl.BlockSpec((M, _BN, XW), lambda i: (0, i, 0))
    nbspec = pl.BlockSpec((_BN, M * NBRF), lambda i: (i, 0))
    w64 = pl.BlockSpec((F, XW), lambda i: (0, 0))
    w41 = pl.BlockSpec((NBRF, XW), lambda i: (0, 0))
    v128 = pl.BlockSpec((1, XW), lambda i: (0, 0))
    return xspec, gspec, nbspec, w64, w41, v128


def _pass1(x, g, nb2, ws, we, bf):
    xspec, gspec, nbspec, w64, w41, v128 = _conv_specs()
    return pl.pallas_call(
        _p1_body,
        grid=(N // _BN,),
        in_specs=[xspec, gspec, nbspec, w64, w41, v128],
        out_specs=[v128, v128],
        out_shape=[jax.ShapeDtypeStruct((1, XW), jnp.float32)] * 2,
    )(x, g, nb2, ws, we, bf)


def _pass2(x, g, nb2, ws, we, bf, k1, h1):
    xspec, gspec, nbspec, w64, w41, v128 = _conv_specs()
    return pl.pallas_call(
        _p2_body,
        grid=(N // _BN,),
        in_specs=[xspec, gspec, nbspec, w64, w41, v128, v128, v128],
        out_specs=[pl.BlockSpec((_BN, F), lambda i: (i, 0)),
                   pl.BlockSpec((1, F), lambda i: (0, 0)),
                   pl.BlockSpec((1, F), lambda i: (0, 0))],
        out_shape=[
            jax.ShapeDtypeStruct((N, F), jnp.float32),
            jax.ShapeDtypeStruct((1, F), jnp.float32),
            jax.ShapeDtypeStruct((1, F), jnp.float32),
        ],
    )(x, g, nb2, ws, we, bf, k1, h1)


_BP3 = 4000


def _p3y_body(x_ref, ns_ref, k_ref, h_ref, wn_ref, x_out, y_out):
    res = jax.nn.softplus(x_ref[...] + ns_ref[...] * k_ref[...] + h_ref[...])
    x_out[...] = res
    y_out[...] = jnp.dot(res, wn_ref[...], preferred_element_type=jnp.float32)


def _p3last_body(x_ref, ns_ref, k_ref, h_ref, o_ref):
    res = jax.nn.softplus(x_ref[...] + ns_ref[...] * k_ref[...] + h_ref[...])
    b = res.shape[0]
    o_ref[...] = jnp.concatenate(
        [res, jnp.ones((b, 1), jnp.float32),
         jnp.zeros((b, XW - F - 1), jnp.float32)], axis=1)


def _pass3(x, ns, k2, h2, wn_next):
    xspec = pl.BlockSpec((_BP3, F), lambda i: (i, 0))
    wspec = pl.BlockSpec((_BP3, XW), lambda i: (i, 0))
    vspec = pl.BlockSpec((1, F), lambda i: (0, 0))
    if wn_next is None:
        return pl.pallas_call(
            _p3last_body,
            grid=(N // _BP3,),
            in_specs=[xspec, xspec, vspec, vspec],
            out_specs=wspec,
            out_shape=jax.ShapeDtypeStruct((N, XW), jnp.float32),
        )(x, ns, k2, h2)
    return pl.pallas_call(
        _p3y_body,
        grid=(N // _BP3,),
        in_specs=[xspec, xspec, vspec, vspec,
                  pl.BlockSpec((F, XW), lambda i: (0, 0))],
        out_specs=[xspec, wspec],
        out_shape=[jax.ShapeDtypeStruct((N, F), jnp.float32),
                   jax.ShapeDtypeStruct((N, XW), jnp.float32)],
    )(x, ns, k2, h2, wn_next)


_SPAD = 2048
_SW = 3200
_SCH = 128
_SCHUNKS = 25
_NPADS = _SW * _NW
_STRIPE = _SPAD // 16


def _seg_body(x_hbm, seg_hbm, zsum_hbm, osum_hbm, xv, segv, acc):
    cid = lax.axis_index("c")
    sid = lax.axis_index("s")
    wid = sid * 2 + cid
    stripe = sid * _STRIPE
    pltpu.sync_copy(zsum_hbm.at[pl.ds(stripe, _STRIPE)],
                    acc.at[pl.ds(stripe, _STRIPE)])
    plsc.subcore_barrier()

    def step(k, carry):
        off = wid * _SW + k * _SCH
        pltpu.sync_copy(seg_hbm.at[pl.ds(off, _SCH)], segv)
        pltpu.sync_copy(x_hbm.at[pl.ds(off, _SCH)], xv)
        pltpu.sync_copy(xv, acc.at[segv], add=True)
        return carry

    lax.fori_loop(0, _SCHUNKS, step, 0)
    plsc.subcore_barrier()
    pltpu.sync_copy(acc.at[pl.ds(stripe, _STRIPE)],
                    osum_hbm.at[cid, pl.ds(stripe, _STRIPE)])


def _sc_segment(x_pad, seg_pad, zsum):
    mesh = plsc.VectorSubcoreMesh(core_axis_name="c", subcore_axis_name="s")
    f = pl.kernel(
        _seg_body,
        mesh=mesh,
        out_type=jax.ShapeDtypeStruct((2, _SPAD, XW), jnp.float32),
        scratch_types=[
            pltpu.VMEM((_SCH, XW), jnp.float32),
            pltpu.VMEM((_SCH,), jnp.int32),
            pltpu.VMEM_SHARED((_SPAD, XW), jnp.float32),
        ],
    )
    return f(x_pad, seg_pad, zsum)


def _pool_body(s_ref, wp_ref, bp_ref, o_ref):
    s = s_ref[0, :, :F] + s_ref[1, :, :F]
    c = s_ref[0, :, F:F + 1] + s_ref[1, :, F:F + 1]
    mean = s / jnp.maximum(c, 1.0)
    out = jnp.dot(mean, wp_ref[...], preferred_element_type=jnp.float32) \
        + bp_ref[...]
    o_ref[...] = jnp.maximum(out, 0.0)[:NSEG]


def _pool(sums, wp, bp):
    return pl.pallas_call(
        _pool_body,
        grid=(1,),
        in_specs=[
            pl.BlockSpec((2, _SPAD, XW), lambda i: (0, 0, 0)),
            pl.BlockSpec((F, F), lambda i: (0, 0)),
            pl.BlockSpec((1, F), lambda i: (0, 0)),
        ],
        out_specs=pl.BlockSpec((NSEG, F), lambda i: (0, 0)),
        out_shape=jax.ShapeDtypeStruct((NSEG, F), jnp.float32),
    )(sums, wp, bp.reshape(1, F))


def kernel(atom_fea, nbr_fea, nbr_fea_idx, crystal_atom_idx, params):
    p = params
    convs = p['convs']
    x, y = _embed(atom_fea, p['W_embed'], p['b_embed'],
                  convs[0]['W_full'][F:2 * F])

    idx_t = jnp.transpose(nbr_fea_idx).astype(jnp.int32)
    idx_pad = jnp.pad(idx_t, ((0, 0), (0, _NPADG - N))).reshape(-1)
    idx_pad = jnp.pad(idx_pad, (0, _IDXPAD - _BPAD))
    nb2 = nbr_fea.reshape(N, M * NBRF)

    for li, pc in enumerate(convs):
        wf = pc['W_full']
        ws = wf[:F]
        we = wf[2 * F:].astype(jnp.bfloat16)
        bf = pc['b_full'][None, :]

        g = _sc_gather(y, idx_pad).reshape(M, _NPADG, XW)
        s1, q1 = _pass1(x, g, nb2, ws, we, bf)
        cnt = float(N * M)
        mu = s1 / cnt
        var = q1 / cnt - mu * mu
        k1 = pc['bn1_g'][None, :] * jax.lax.rsqrt(var + 1e-5)
        h1 = pc['bn1_b'][None, :] - mu * k1

        ns, s2, q2 = _pass2(x, g, nb2, ws, we, bf, k1, h1)
        mu2 = s2 / float(N)
        var2 = q2 / float(N) - mu2 * mu2
        k2 = pc['bn2_g'][None, :] * jax.lax.rsqrt(var2 + 1e-5)
        h2 = pc['bn2_b'][None, :] - mu2 * k2
        wn_next = (convs[li + 1]['W_full'][F:2 * F]
                   if li + 1 < len(convs) else None)
        if wn_next is None:
            x = _pass3(x, ns, k2, h2, None)
        else:
            x, y = _pass3(x, ns, k2, h2, wn_next)

    x_pad = jnp.pad(x, ((0, _NPADS - N), (0, 0)))
    seg_pad = jnp.pad(crystal_atom_idx.astype(jnp.int32), (0, _NPADS - N),
                      constant_values=NSEG)
    zsum = jnp.zeros((_SPAD, XW), jnp.float32)
    sums = _sc_segment(x_pad, seg_pad, zsum)
    return _pool(sums, p['W_pool'], p['b_pool'])


# speedup vs baseline: 1.0879x; 1.0879x over previous
import jax
import jax.numpy as jnp
from jax import lax
from jax.experimental import pallas as pl
from jax.experimental.pallas import tpu as pltpu
from jax.experimental.pallas import tpu_sc as plsc

F = 64
XW = 128
NBRF = 41
M = 12
N = 100000
NSEG = 2000

_BE = 2000


def _embed_body(a_ref, w_ref, b_ref, wn_ref, x_ref, y_ref):
    res = jnp.dot(a_ref[...], w_ref[...],
                  preferred_element_type=jnp.float32) + b_ref[...]
    x_ref[...] = res
    y_ref[...] = jnp.dot(res, wn_ref[...],
                         preferred_element_type=jnp.float32)


def _embed(atom_fea, w, b, wn):
    n, orig = atom_fea.shape
    return pl.pallas_call(
        _embed_body,
        grid=(n // _BE,),
        in_specs=[
            pl.BlockSpec((_BE, orig), lambda i: (i, 0)),
            pl.BlockSpec((orig, F), lambda i: (0, 0)),
            pl.BlockSpec((1, F), lambda i: (0, 0)),
            pl.BlockSpec((F, XW), lambda i: (0, 0)),
        ],
        out_specs=[pl.BlockSpec((_BE, F), lambda i: (i, 0)),
                   pl.BlockSpec((_BE, XW), lambda i: (i, 0))],
        out_shape=[jax.ShapeDtypeStruct((n, F), jnp.float32),
                   jax.ShapeDtypeStruct((n, XW), jnp.float32)],
    )(atom_fea, w, b.reshape(1, F), wn)


_NW = 32
_GCH = 160
_GCHUNKS = 240
_WPER = _GCH * _GCHUNKS
_NPADG = 102400
_BPAD = _WPER * _NW
_IDXPAD = _BPAD


def _gather_body(table, idx_hbm, out_hbm, idx_all, rows_v,
                 g0, g1, g2, w0, w1, w2):
    wid = lax.axis_index("s") * 2 + lax.axis_index("c")
    base = wid * _WPER
    gsem = (g0, g1, g2)
    wsem = (w0, w1, w2)
    pltpu.sync_copy(idx_hbm.at[pl.ds(base, _WPER)], idx_all)

    def gather_copy(i, b):
        return pltpu.make_async_copy(
            table.at[idx_all.at[pl.ds(i * _GCH, _GCH)]],
            rows_v.at[pl.ds(b * _GCH, _GCH)], gsem[b])

    def write_copy(i, b):
        return pltpu.make_async_copy(
            rows_v.at[pl.ds(b * _GCH, _GCH)],
            out_hbm.at[pl.ds(base + i * _GCH, _GCH)], wsem[b])

    gather_copy(0, 0).start()
    gather_copy(1, 1).start()

    def step(j, carry):
        for b in range(3):
            i = 3 * j + b
            gather_copy(i, b).wait()
            write_copy(i, b).start()
            c = (b + 2) % 3

            @pl.when(i + 2 < _GCHUNKS)
            def _():
                @pl.when(i >= 1)
                def _():
                    write_copy(i - 1, c).wait()
                gather_copy(i + 2, c).start()
        return carry

    lax.fori_loop(0, _GCHUNKS // 3, step, 0)
    for t in (_GCHUNKS - 3, _GCHUNKS - 2, _GCHUNKS - 1):
        write_copy(t, t % 3).wait()


def _sc_gather(table, idx_pad):
    mesh = plsc.VectorSubcoreMesh(core_axis_name="c", subcore_axis_name="s")
    f = pl.kernel(
        _gather_body,
        mesh=mesh,
        out_type=jax.ShapeDtypeStruct((_BPAD, XW), jnp.float32),
        scratch_types=[
            pltpu.VMEM((_WPER,), jnp.int32),
            pltpu.VMEM((3 * _GCH, XW), jnp.float32),
            pltpu.SemaphoreType.DMA,
            pltpu.SemaphoreType.DMA,
            pltpu.SemaphoreType.DMA,
            pltpu.SemaphoreType.DMA,
            pltpu.SemaphoreType.DMA,
            pltpu.SemaphoreType.DMA,
        ],
    )
    return f(table, idx_pad)


_BN = 2000


def _p1_body(x_ref, g_ref, nb_ref, ws, we, bf, s_ref, q_ref):
    i = pl.program_id(0)
    s = jnp.dot(x_ref[...], ws[...],
                preferred_element_type=jnp.float32) + bf[...]
    acc_s = jnp.zeros((1, XW), jnp.float32)
    acc_q = jnp.zeros((1, XW), jnp.float32)
    for m in range(M):
        nb = nb_ref[:, pl.ds(NBRF * m, NBRF)].astype(jnp.bfloat16)
        z = s + g_ref[m] \
            + jnp.dot(nb, we[...], preferred_element_type=jnp.float32)
        acc_s += jnp.sum(z, axis=0, keepdims=True)
        acc_q += jnp.sum(z * z, axis=0, keepdims=True)

    @pl.when(i == 0)
    def _():
        s_ref[...] = jnp.zeros_like(s_ref)
        q_ref[...] = jnp.zeros_like(q_ref)

    s_ref[...] += acc_s
    q_ref[...] += acc_q


def _p2_body(x_ref, g_ref, nb_ref, ws, we, bf, k1, h1,
             ns_ref, s2_ref, q2_ref):
    i = pl.program_id(0)
    s = jnp.dot(x_ref[...], ws[...],
                preferred_element_type=jnp.float32) + bf[...]
    acc = jnp.zeros((_BN, F), jnp.float32)
    for m in range(M):
        nb = nb_ref[:, pl.ds(NBRF * m, NBRF)].astype(jnp.bfloat16)
        z = s + g_ref[m] \
            + jnp.dot(nb, we[...], preferred_element_type=jnp.float32)
        z = z * k1[...] + h1[...]
        acc += jax.nn.sigmoid(z[:, :F]) * jax.nn.softplus(z[:, F:])
    ns_ref[...] = acc

    @pl.when(i == 0)
    def _():
        s2_ref[...] = jnp.zeros_like(s2_ref)
        q2_ref[...] = jnp.zeros_like(q2_ref)

    s2_ref[...] += jnp.sum(acc, axis=0, keepdims=True)
    q2_ref[...] += jnp.sum(acc * acc, axis=0, keepdims=True)


def _conv_specs():
    xspec = pl.BlockSpec((_BN, F), lambda i: (i, 0))
    gspec = pl.BlockSpec((M, _BN, XW), lambda i: (0, i, 0))
    nbspec = pl.BlockSpec((_BN, M * NBRF), lambda i: (i, 0))
    w64 = pl.BlockSpec((F, XW), lambda i: (0, 0))
    w41 = pl.BlockSpec((NBRF, XW), lambda i: (0, 0))
    v128 = pl.BlockSpec((1, XW), lambda i: (0, 0))
    return xspec, gspec, nbspec, w64, w41, v128


def _pass1(x, g, nb2, ws, we, bf):
    xspec, gspec, nbspec, w64, w41, v128 = _conv_specs()
    return pl.pallas_call(
        _p1_body,
        grid=(N // _BN,),
        in_specs=[xspec, gspec, nbspec, w64, w41, v128],
        out_specs=[v128, v128],
        out_shape=[jax.ShapeDtypeStruct((1, XW), jnp.float32)] * 2,
    )(x, g, nb2, ws, we, bf)


def _pass2(x, g, nb2, ws, we, bf, k1, h1):
    xspec, gspec, nbspec, w64, w41, v128 = _conv_specs()
    return pl.pallas_call(
        _p2_body,
        grid=(N // _BN,),
        in_specs=[xspec, gspec, nbspec, w64, w41, v128, v128, v128],
        out_specs=[pl.BlockSpec((_BN, F), lambda i: (i, 0)),
                   pl.BlockSpec((1, F), lambda i: (0, 0)),
                   pl.BlockSpec((1, F), lambda i: (0, 0))],
        out_shape=[
            jax.ShapeDtypeStruct((N, F), jnp.float32),
            jax.ShapeDtypeStruct((1, F), jnp.float32),
            jax.ShapeDtypeStruct((1, F), jnp.float32),
        ],
    )(x, g, nb2, ws, we, bf, k1, h1)


_BP3 = 4000


def _p3y_body(x_ref, ns_ref, k_ref, h_ref, wn_ref, x_out, y_out):
    res = jax.nn.softplus(x_ref[...] + ns_ref[...] * k_ref[...] + h_ref[...])
    x_out[...] = res
    y_out[...] = jnp.dot(res, wn_ref[...], preferred_element_type=jnp.float32)


def _p3last_body(x_ref, ns_ref, k_ref, h_ref, o_ref):
    res = jax.nn.softplus(x_ref[...] + ns_ref[...] * k_ref[...] + h_ref[...])
    b = res.shape[0]
    o_ref[...] = jnp.concatenate(
        [res, jnp.ones((b, 1), jnp.float32),
         jnp.zeros((b, XW - F - 1), jnp.float32)], axis=1)


def _pass3(x, ns, k2, h2, wn_next):
    xspec = pl.BlockSpec((_BP3, F), lambda i: (i, 0))
    wspec = pl.BlockSpec((_BP3, XW), lambda i: (i, 0))
    vspec = pl.BlockSpec((1, F), lambda i: (0, 0))
    if wn_next is None:
        return pl.pallas_call(
            _p3last_body,
            grid=(N // _BP3,),
            in_specs=[xspec, xspec, vspec, vspec],
            out_specs=wspec,
            out_shape=jax.ShapeDtypeStruct((N, XW), jnp.float32),
        )(x, ns, k2, h2)
    return pl.pallas_call(
        _p3y_body,
        grid=(N // _BP3,),
        in_specs=[xspec, xspec, vspec, vspec,
                  pl.BlockSpec((F, XW), lambda i: (0, 0))],
        out_specs=[xspec, wspec],
        out_shape=[jax.ShapeDtypeStruct((N, F), jnp.float32),
                   jax.ShapeDtypeStruct((N, XW), jnp.float32)],
    )(x, ns, k2, h2, wn_next)


_SPAD = 2048
_SW = 3200
_SCH = 128
_SCHUNKS = 25
_NPADS = _SW * _NW
_STRIPE = _SPAD // 16


def _seg_body(x_hbm, seg_hbm, zsum_hbm, osum_hbm, xv, segv, acc):
    cid = lax.axis_index("c")
    sid = lax.axis_index("s")
    wid = sid * 2 + cid
    stripe = sid * _STRIPE
    pltpu.sync_copy(zsum_hbm.at[pl.ds(stripe, _STRIPE)],
                    acc.at[pl.ds(stripe, _STRIPE)])
    plsc.subcore_barrier()

    def step(k, carry):
        off = wid * _SW + k * _SCH
        pltpu.sync_copy(seg_hbm.at[pl.ds(off, _SCH)], segv)
        pltpu.sync_copy(x_hbm.at[pl.ds(off, _SCH)], xv)
        pltpu.sync_copy(xv, acc.at[segv], add=True)
        return carry

    lax.fori_loop(0, _SCHUNKS, step, 0)
    plsc.subcore_barrier()
    pltpu.sync_copy(acc.at[pl.ds(stripe, _STRIPE)],
                    osum_hbm.at[cid, pl.ds(stripe, _STRIPE)])


def _sc_segment(x_pad, seg_pad, zsum):
    mesh = plsc.VectorSubcoreMesh(core_axis_name="c", subcore_axis_name="s")
    f = pl.kernel(
        _seg_body,
        mesh=mesh,
        out_type=jax.ShapeDtypeStruct((2, _SPAD, XW), jnp.float32),
        scratch_types=[
            pltpu.VMEM((_SCH, XW), jnp.float32),
            pltpu.VMEM((_SCH,), jnp.int32),
            pltpu.VMEM_SHARED((_SPAD, XW), jnp.float32),
        ],
    )
    return f(x_pad, seg_pad, zsum)


def _pool_body(s_ref, wp_ref, bp_ref, o_ref):
    s = s_ref[0, :, :F] + s_ref[1, :, :F]
    c = s_ref[0, :, F:F + 1] + s_ref[1, :, F:F + 1]
    mean = s / jnp.maximum(c, 1.0)
    out = jnp.dot(mean, wp_ref[...], preferred_element_type=jnp.float32) \
        + bp_ref[...]
    o_ref[...] = jnp.maximum(out, 0.0)[:NSEG]


def _pool(sums, wp, bp):
    return pl.pallas_call(
        _pool_body,
        grid=(1,),
        in_specs=[
            pl.BlockSpec((2, _SPAD, XW), lambda i: (0, 0, 0)),
            pl.BlockSpec((F, F), lambda i: (0, 0)),
            pl.BlockSpec((1, F), lambda i: (0, 0)),
        ],
        out_specs=pl.BlockSpec((NSEG, F), lambda i: (0, 0)),
        out_shape=jax.ShapeDtypeStruct((NSEG, F), jnp.float32),
    )(sums, wp, bp.reshape(1, F))


def kernel(atom_fea, nbr_fea, nbr_fea_idx, crystal_atom_idx, params):
    p = params
    convs = p['convs']
    x, y = _embed(atom_fea, p['W_embed'], p['b_embed'],
                  convs[0]['W_full'][F:2 * F])

    idx_t = jnp.transpose(nbr_fea_idx).astype(jnp.int32)
    idx_pad = jnp.pad(idx_t, ((0, 0), (0, _NPADG - N))).reshape(-1)
    idx_pad = jnp.pad(idx_pad, (0, _IDXPAD - _BPAD))
    nb2 = nbr_fea.reshape(N, M * NBRF)

    for li, pc in enumerate(convs):
        wf = pc['W_full']
        ws = wf[:F]
        we = wf[2 * F:].astype(jnp.bfloat16)
        bf = pc['b_full'][None, :]

        g = _sc_gather(y, idx_pad).reshape(M, _NPADG, XW)
        s1, q1 = _pass1(x, g, nb2, ws, we, bf)
        cnt = float(N * M)
        mu = s1 / cnt
        var = q1 / cnt - mu * mu
        k1 = pc['bn1_g'][None, :] * jax.lax.rsqrt(var + 1e-5)
        h1 = pc['bn1_b'][None, :] - mu * k1

        ns, s2, q2 = _pass2(x, g, nb2, ws, we, bf, k1, h1)
        mu2 = s2 / float(N)
        var2 = q2 / float(N) - mu2 * mu2
        k2 = pc['bn2_g'][None, :] * jax.lax.rsqrt(var2 + 1e-5)
        h2 = pc['bn2_b'][None, :] - mu2 * k2
        wn_next = (convs[li + 1]['W_full'][F:2 * F]
                   if li + 1 < len(convs) else None)
        if wn_next is None:
            x = _pass3(x, ns, k2, h2, None)
        else:
            x, y = _pass3(x, ns, k2, h2, wn_next)

    x_pad = jnp.pad(x, ((0, _NPADS - N), (0, 0)))
    seg_pad = jnp.pad(crystal_atom_idx.astype(jnp.int32), (0, _NPADS - N),
                      constant_values=NSEG)
    zsum = jnp.zeros((_SPAD, XW), jnp.float32)
    sums = _sc_segment(x_pad, seg_pad, zsum)
    return _pool(sums, p['W_pool'], p['b_pool'])
